# baseline (speedup 1.0000x reference)
import functools

import jax
import jax.numpy as jnp
from jax import lax
from jax.experimental import pallas as pl
from jax.experimental.pallas import tpu as pltpu
from jax.experimental.pallas import tpu_sc as plsc

N_NODES = 100000
D_FEAT = 128
N_GRAPHS = 64
EPS = 1e-7

ROW_BLOCK = 4000
N_BLOCKS = N_NODES // ROW_BLOCK

OFF_BLOCK = 20000
OFF_BLOCKS = N_NODES // OFF_BLOCK

NUM_CORES = 2
NUM_SUBCORES = 16
LANES = 16
NUM_WORKERS = NUM_CORES * NUM_SUBCORES
GRAPHS_PER_WORKER = N_GRAPHS // NUM_WORKERS
SC_BLOCK = 320
FCHUNKS = D_FEAT // LANES


def _offsets_body(ids_ref, off_ref, acc):
    i = pl.program_id(0)

    @pl.when(i == 0)
    def _init():
        acc[...] = jnp.zeros_like(acc)

    ids = ids_ref[0, 0, :]
    gi = jax.lax.broadcasted_iota(jnp.int32, (OFF_BLOCK, 128), 1)
    cmp = (ids[:, None] < gi).astype(jnp.int32)
    acc[...] += jnp.sum(cmp, axis=0, keepdims=True)

    @pl.when(i == OFF_BLOCKS - 1)
    def _fin():
        off_ref[...] = acc[...]


def _sc_stats_body(x_hbm, off_hbm, sums_hbm, sumsq_hbm, cnt_hbm,
                   off_v, xbuf0, xbuf1, xbuf2, stage, sem0, sem1, sem2):
    wid = lax.axis_index("s") * NUM_CORES + lax.axis_index("c")
    pltpu.sync_copy(off_hbm, off_v)
    iota16 = lax.iota(jnp.int32, LANES)
    zeros16 = jnp.zeros((LANES,), jnp.float32)
    xbufs = (xbuf0, xbuf1, xbuf2)
    sems = (sem0, sem1, sem2)

    def _src(k):
        rb_c = pl.multiple_of(
            jnp.minimum(k * SC_BLOCK, N_NODES - SC_BLOCK), 8)
        return x_hbm.at[pl.ds(rb_c, SC_BLOCK), :]

    for j in range(GRAPHS_PER_WORKER):
        g = wid * GRAPHS_PER_WORKER + j
        g_al = pl.multiple_of((g // 8) * 8, 8)
        vals = off_v[pl.ds(g_al, LANES)]
        tloc = g - g_al
        start = jnp.max(jnp.where(iota16 == tloc, vals, 0))
        end = jnp.max(jnp.where(iota16 == tloc + 1, vals, 0))
        k0 = start // SC_BLOCK
        k1 = (end + SC_BLOCK - 1) // SC_BLOCK
        ntrips = (k1 - k0 + 1) // 3

        pltpu.async_copy(_src(k0), xbuf0, sem0)
        pltpu.async_copy(_src(k0 + 1), xbuf1, sem1)
        pltpu.async_copy(_src(k0 + 2), xbuf2, sem2)

        def _accum_block(k, xbuf, carry):
            sums, sumsq, cntv = carry
            rb = k * SC_BLOCK
            rb_c = jnp.minimum(rb, N_NODES - SC_BLOCK)
            lo = jnp.maximum(start, rb) - rb_c
            hi = jnp.minimum(rb + SC_BLOCK, end) - rb_c
            nvalid = jnp.maximum(hi - lo, 0).astype(jnp.float32)
            cntv = cntv + lax.broadcast_in_dim(nvalid, (LANES,), ())

            def full_row(r, rcarry):
                rsums, rsumsq = rcarry
                new_s = []
                new_q = []
                for f in range(FCHUNKS):
                    v = xbuf[r, pl.ds(LANES * f, LANES)]
                    new_s.append(rsums[f] + v)
                    new_q.append(rsumsq[f] + v * v)
                return tuple(new_s), tuple(new_q)

            def masked_row(r, rcarry):
                rsums, rsumsq = rcarry
                valid = (r >= lo) & (r < hi)
                wf = jnp.where(valid, 1.0, 0.0)
                wfv = lax.broadcast_in_dim(wf, (LANES,), ())
                new_s = []
                new_q = []
                for f in range(FCHUNKS):
                    v = xbuf[r, pl.ds(LANES * f, LANES)]
                    vw = v * wfv
                    new_s.append(rsums[f] + vw)
                    new_q.append(rsumsq[f] + vw * v)
                return tuple(new_s), tuple(new_q)

            full = (lo == 0) & (hi == SC_BLOCK)
            sums, sumsq = lax.cond(
                full,
                lambda c: lax.fori_loop(0, SC_BLOCK, full_row, c, unroll=4),
                lambda c: lax.fori_loop(0, SC_BLOCK, masked_row, c, unroll=4),
                (sums, sumsq),
            )
            return sums, sumsq, cntv

        def trip_body(p, carry):
            k = k0 + 3 * p
            for b in range(3):
                pltpu.make_async_copy(_src(k + b), xbufs[b], sems[b]).wait()
                carry = _accum_block(k + b, xbufs[b], carry)
                pltpu.async_copy(_src(k + b + 3), xbufs[b], sems[b])
            return carry

        init = (tuple(zeros16 for _ in range(FCHUNKS)),
                tuple(zeros16 for _ in range(FCHUNKS)),
                zeros16)
        carry = lax.fori_loop(0, ntrips, trip_body, init)
        k_t = k0 + 3 * ntrips
        pltpu.make_async_copy(_src(k_t), xbuf0, sem0).wait()
        sums, sumsq, cntv = _accum_block(k_t, xbuf0, carry)
        pltpu.make_async_copy(_src(k_t + 1), xbuf1, sem1).wait()
        pltpu.make_async_copy(_src(k_t + 2), xbuf2, sem2).wait()

        for f in range(FCHUNKS):
            stage[pl.ds(LANES * f, LANES)] = sums[f]
        pltpu.sync_copy(stage, sums_hbm.at[g])
        for f in range(FCHUNKS):
            stage[pl.ds(LANES * f, LANES)] = sumsq[f]
        pltpu.sync_copy(stage, sumsq_hbm.at[g])
        for f in range(FCHUNKS):
            stage[pl.ds(LANES * f, LANES)] = cntv
        pltpu.sync_copy(stage, cnt_hbm.at[g])


_sc_stats = functools.partial(
    pl.kernel,
    _sc_stats_body,
    mesh=plsc.VectorSubcoreMesh(core_axis_name="c", subcore_axis_name="s"),
    out_type=[
        jax.ShapeDtypeStruct((N_GRAPHS, D_FEAT), jnp.float32),
        jax.ShapeDtypeStruct((N_GRAPHS, D_FEAT), jnp.float32),
        jax.ShapeDtypeStruct((N_GRAPHS, D_FEAT), jnp.float32),
    ],
    scratch_types=[
        pltpu.VMEM((128,), jnp.int32),
        pltpu.VMEM((SC_BLOCK, D_FEAT), jnp.float32),
        pltpu.VMEM((SC_BLOCK, D_FEAT), jnp.float32),
        pltpu.VMEM((SC_BLOCK, D_FEAT), jnp.float32),
        pltpu.VMEM((D_FEAT,), jnp.float32),
        pltpu.SemaphoreType.DMA,
        pltpu.SemaphoreType.DMA,
        pltpu.SemaphoreType.DMA,
    ],
    compiler_params=pltpu.CompilerParams(
        needs_layout_passes=False, use_tc_tiling_on_sc=False
    ),
)


def _apply_body(ids_ref, x_ref, sums_ref, sumsq_ref, cnt_ref,
                a_ref, b_ref, g_ref, o_ref, scale_s, bias_s):
    i = pl.program_id(0)

    @pl.when(i == 0)
    def _finalize():
        cnt = jnp.maximum(cnt_ref[...][:, 0:1], 1.0)
        inv_n = 1.0 / cnt
        mean = sums_ref[...] * inv_n
        msq = sumsq_ref[...] * inv_n
        a = a_ref[...]
        var = msq - mean * mean * (2.0 * a - a * a)
        var = jnp.maximum(var, 0.0)
        inv = 1.0 / (jnp.sqrt(var) + EPS)
        scale = inv * g_ref[...]
        scale_s[...] = scale
        bias_s[...] = b_ref[...] - a * mean * scale

    ids = ids_ref[0, 0, :]
    gi = jax.lax.broadcasted_iota(jnp.int32, (ROW_BLOCK, N_GRAPHS), 1)
    oh = (ids[:, None] == gi).astype(jnp.float32)
    dn = (((1,), (0,)), ((), ()))
    s = jax.lax.dot_general(oh, scale_s[...], dn, preferred_element_type=jnp.float32)
    b = jax.lax.dot_general(oh, bias_s[...], dn, preferred_element_type=jnp.float32)
    o_ref[...] = x_ref[...] * s + b


@jax.jit
def kernel(node_features, node_to_graph_map, alpha, beta, gamma):
    ids3o = node_to_graph_map.reshape(OFF_BLOCKS, 1, OFF_BLOCK)
    off2 = pl.pallas_call(
        _offsets_body,
        grid=(OFF_BLOCKS,),
        in_specs=[pl.BlockSpec((1, 1, OFF_BLOCK), lambda i: (i, 0, 0))],
        out_specs=pl.BlockSpec((1, 128), lambda i: (0, 0)),
        out_shape=jax.ShapeDtypeStruct((1, 128), jnp.int32),
        scratch_shapes=[pltpu.VMEM((1, 128), jnp.int32)],
    )(ids3o)
    off_pad = off2.reshape(128)

    sums, sumsq, cnt = _sc_stats()(node_features, off_pad)

    ids3 = node_to_graph_map.reshape(N_BLOCKS, 1, ROW_BLOCK)
    a2 = alpha.reshape(1, D_FEAT)
    b2 = beta.reshape(1, D_FEAT)
    g2 = gamma.reshape(1, D_FEAT)

    out = pl.pallas_call(
        _apply_body,
        grid=(N_BLOCKS,),
        in_specs=[
            pl.BlockSpec((1, 1, ROW_BLOCK), lambda i: (i, 0, 0)),
            pl.BlockSpec((ROW_BLOCK, D_FEAT), lambda i: (i, 0)),
            pl.BlockSpec((N_GRAPHS, D_FEAT), lambda i: (0, 0)),
            pl.BlockSpec((N_GRAPHS, D_FEAT), lambda i: (0, 0)),
            pl.BlockSpec((N_GRAPHS, D_FEAT), lambda i: (0, 0)),
            pl.BlockSpec((1, D_FEAT), lambda i: (0, 0)),
            pl.BlockSpec((1, D_FEAT), lambda i: (0, 0)),
            pl.BlockSpec((1, D_FEAT), lambda i: (0, 0)),
        ],
        out_specs=pl.BlockSpec((ROW_BLOCK, D_FEAT), lambda i: (i, 0)),
        out_shape=jax.ShapeDtypeStruct((N_NODES, D_FEAT), jnp.float32),
        scratch_shapes=[
            pltpu.VMEM((N_GRAPHS, D_FEAT), jnp.float32),
            pltpu.VMEM((N_GRAPHS, D_FEAT), jnp.float32),
        ],
    )(ids3, node_features, sums, sumsq, cnt, a2, b2, g2)
    return out


# speedup vs baseline: 1.0158x; 1.0158x over previous
import functools

import jax
import jax.numpy as jnp
from jax import lax
from jax.experimental import pallas as pl
from jax.experimental.pallas import tpu as pltpu
from jax.experimental.pallas import tpu_sc as plsc

N_NODES = 100000
D_FEAT = 128
N_GRAPHS = 64
EPS = 1e-7

ROW_BLOCK = 4000
N_BLOCKS = N_NODES // ROW_BLOCK

OFF_BLOCK = 10000
OFF_BLOCKS = N_NODES // OFF_BLOCK

NUM_CORES = 2
NUM_SUBCORES = 16
LANES = 16
NUM_WORKERS = NUM_CORES * NUM_SUBCORES
GRAPHS_PER_WORKER = N_GRAPHS // NUM_WORKERS
SC_BLOCK = 256
FCHUNKS = D_FEAT // LANES


def _offsets_body(ids_ref, off_ref, acc):
    i = pl.program_id(0)

    @pl.when(i == 0)
    def _init():
        acc[...] = jnp.zeros_like(acc)

    ids = ids_ref[0, 0, :]
    gi = jax.lax.broadcasted_iota(jnp.int32, (OFF_BLOCK, 128), 1)
    cmp = (ids[:, None] < gi).astype(jnp.int32)
    acc[...] += jnp.sum(cmp, axis=0, keepdims=True)

    @pl.when(i == OFF_BLOCKS - 1)
    def _fin():
        off_ref[...] = acc[...]


def _sc_stats_body(x_hbm, off_hbm, sums_hbm, sumsq_hbm, cnt_hbm,
                   off_v, xbuf0, xbuf1, xbuf2, stage, sem0, sem1, sem2):
    wid = lax.axis_index("s") * NUM_CORES + lax.axis_index("c")
    pltpu.sync_copy(off_hbm, off_v)
    iota16 = lax.iota(jnp.int32, LANES)
    zeros16 = jnp.zeros((LANES,), jnp.float32)
    xbufs = (xbuf0, xbuf1, xbuf2)
    sems = (sem0, sem1, sem2)

    def _src(k):
        rb_c = pl.multiple_of(
            jnp.minimum(k * SC_BLOCK, N_NODES - SC_BLOCK), 8)
        return x_hbm.at[pl.ds(rb_c, SC_BLOCK), :]

    for j in range(GRAPHS_PER_WORKER):
        g = wid * GRAPHS_PER_WORKER + j
        g_al = pl.multiple_of((g // 8) * 8, 8)
        vals = off_v[pl.ds(g_al, LANES)]
        tloc = g - g_al
        start = jnp.max(jnp.where(iota16 == tloc, vals, 0))
        end = jnp.max(jnp.where(iota16 == tloc + 1, vals, 0))
        k0 = start // SC_BLOCK
        k1 = (end + SC_BLOCK - 1) // SC_BLOCK
        ntrips = (k1 - k0 + 1) // 3

        pltpu.async_copy(_src(k0), xbuf0, sem0)
        pltpu.async_copy(_src(k0 + 1), xbuf1, sem1)
        pltpu.async_copy(_src(k0 + 2), xbuf2, sem2)

        def _accum_block(k, xbuf, carry):
            sums, sumsq, cntv = carry
            rb = k * SC_BLOCK
            rb_c = jnp.minimum(rb, N_NODES - SC_BLOCK)
            lo = jnp.maximum(start, rb) - rb_c
            hi = jnp.minimum(rb + SC_BLOCK, end) - rb_c
            nvalid = jnp.maximum(hi - lo, 0).astype(jnp.float32)
            cntv = cntv + lax.broadcast_in_dim(nvalid, (LANES,), ())

            def full_row(r, rcarry):
                rsums, rsumsq = rcarry
                new_s = []
                new_q = []
                for f in range(FCHUNKS):
                    v = xbuf[r, pl.ds(LANES * f, LANES)]
                    new_s.append(rsums[f] + v)
                    new_q.append(rsumsq[f] + v * v)
                return tuple(new_s), tuple(new_q)

            def masked_row(r, rcarry):
                rsums, rsumsq = rcarry
                valid = (r >= lo) & (r < hi)
                wf = jnp.where(valid, 1.0, 0.0)
                wfv = lax.broadcast_in_dim(wf, (LANES,), ())
                new_s = []
                new_q = []
                for f in range(FCHUNKS):
                    v = xbuf[r, pl.ds(LANES * f, LANES)]
                    vw = v * wfv
                    new_s.append(rsums[f] + vw)
                    new_q.append(rsumsq[f] + vw * v)
                return tuple(new_s), tuple(new_q)

            full = (lo == 0) & (hi == SC_BLOCK)
            sums, sumsq = lax.cond(
                full,
                lambda c: lax.fori_loop(0, SC_BLOCK, full_row, c, unroll=4),
                lambda c: lax.fori_loop(0, SC_BLOCK, masked_row, c, unroll=4),
                (sums, sumsq),
            )
            return sums, sumsq, cntv

        def trip_body(p, carry):
            k = k0 + 3 * p
            for b in range(3):
                pltpu.make_async_copy(_src(k + b), xbufs[b], sems[b]).wait()
                carry = _accum_block(k + b, xbufs[b], carry)
                pltpu.async_copy(_src(k + b + 3), xbufs[b], sems[b])
            return carry

        init = (tuple(zeros16 for _ in range(FCHUNKS)),
                tuple(zeros16 for _ in range(FCHUNKS)),
                zeros16)
        carry = lax.fori_loop(0, ntrips, trip_body, init)
        k_t = k0 + 3 * ntrips
        pltpu.make_async_copy(_src(k_t), xbuf0, sem0).wait()
        sums, sumsq, cntv = _accum_block(k_t, xbuf0, carry)
        pltpu.make_async_copy(_src(k_t + 1), xbuf1, sem1).wait()
        pltpu.make_async_copy(_src(k_t + 2), xbuf2, sem2).wait()

        for f in range(FCHUNKS):
            stage[pl.ds(LANES * f, LANES)] = sums[f]
        pltpu.sync_copy(stage, sums_hbm.at[g])
        for f in range(FCHUNKS):
            stage[pl.ds(LANES * f, LANES)] = sumsq[f]
        pltpu.sync_copy(stage, sumsq_hbm.at[g])
        for f in range(FCHUNKS):
            stage[pl.ds(LANES * f, LANES)] = cntv
        pltpu.sync_copy(stage, cnt_hbm.at[g])


_sc_stats = functools.partial(
    pl.kernel,
    _sc_stats_body,
    mesh=plsc.VectorSubcoreMesh(core_axis_name="c", subcore_axis_name="s"),
    out_type=[
        jax.ShapeDtypeStruct((N_GRAPHS, D_FEAT), jnp.float32),
        jax.ShapeDtypeStruct((N_GRAPHS, D_FEAT), jnp.float32),
        jax.ShapeDtypeStruct((N_GRAPHS, D_FEAT), jnp.float32),
    ],
    scratch_types=[
        pltpu.VMEM((128,), jnp.int32),
        pltpu.VMEM((SC_BLOCK, D_FEAT), jnp.float32),
        pltpu.VMEM((SC_BLOCK, D_FEAT), jnp.float32),
        pltpu.VMEM((SC_BLOCK, D_FEAT), jnp.float32),
        pltpu.VMEM((D_FEAT,), jnp.float32),
        pltpu.SemaphoreType.DMA,
        pltpu.SemaphoreType.DMA,
        pltpu.SemaphoreType.DMA,
    ],
    compiler_params=pltpu.CompilerParams(
        needs_layout_passes=False, use_tc_tiling_on_sc=False
    ),
)


def _apply_body(ids_ref, x_ref, sums_ref, sumsq_ref, cnt_ref,
                a_ref, b_ref, g_ref, o_ref, scale_s, bias_s):
    i = pl.program_id(0)

    @pl.when(i == 0)
    def _finalize():
        cnt = jnp.maximum(cnt_ref[...][:, 0:1], 1.0)
        inv_n = 1.0 / cnt
        mean = sums_ref[...] * inv_n
        msq = sumsq_ref[...] * inv_n
        a = a_ref[...]
        var = msq - mean * mean * (2.0 * a - a * a)
        var = jnp.maximum(var, 0.0)
        inv = 1.0 / (jnp.sqrt(var) + EPS)
        scale = inv * g_ref[...]
        scale_s[...] = scale
        bias_s[...] = b_ref[...] - a * mean * scale

    ids = ids_ref[0, 0, :]
    gi = jax.lax.broadcasted_iota(jnp.int32, (ROW_BLOCK, N_GRAPHS), 1)
    oh = (ids[:, None] == gi).astype(jnp.float32)
    dn = (((1,), (0,)), ((), ()))
    s = jax.lax.dot_general(oh, scale_s[...], dn, preferred_element_type=jnp.float32)
    b = jax.lax.dot_general(oh, bias_s[...], dn, preferred_element_type=jnp.float32)
    o_ref[...] = x_ref[...] * s + b


@jax.jit
def kernel(node_features, node_to_graph_map, alpha, beta, gamma):
    ids3o = node_to_graph_map.reshape(OFF_BLOCKS, 1, OFF_BLOCK)
    off2 = pl.pallas_call(
        _offsets_body,
        grid=(OFF_BLOCKS,),
        in_specs=[pl.BlockSpec((1, 1, OFF_BLOCK), lambda i: (i, 0, 0))],
        out_specs=pl.BlockSpec((1, 128), lambda i: (0, 0)),
        out_shape=jax.ShapeDtypeStruct((1, 128), jnp.int32),
        scratch_shapes=[pltpu.VMEM((1, 128), jnp.int32)],
    )(ids3o)
    off_pad = off2.reshape(128)

    sums, sumsq, cnt = _sc_stats()(node_features, off_pad)

    ids3 = node_to_graph_map.reshape(N_BLOCKS, 1, ROW_BLOCK)
    a2 = alpha.reshape(1, D_FEAT)
    b2 = beta.reshape(1, D_FEAT)
    g2 = gamma.reshape(1, D_FEAT)

    out = pl.pallas_call(
        _apply_body,
        grid=(N_BLOCKS,),
        in_specs=[
            pl.BlockSpec((1, 1, ROW_BLOCK), lambda i: (i, 0, 0)),
            pl.BlockSpec((ROW_BLOCK, D_FEAT), lambda i: (i, 0)),
            pl.BlockSpec((N_GRAPHS, D_FEAT), lambda i: (0, 0)),
            pl.BlockSpec((N_GRAPHS, D_FEAT), lambda i: (0, 0)),
            pl.BlockSpec((N_GRAPHS, D_FEAT), lambda i: (0, 0)),
            pl.BlockSpec((1, D_FEAT), lambda i: (0, 0)),
            pl.BlockSpec((1, D_FEAT), lambda i: (0, 0)),
            pl.BlockSpec((1, D_FEAT), lambda i: (0, 0)),
        ],
        out_specs=pl.BlockSpec((ROW_BLOCK, D_FEAT), lambda i: (i, 0)),
        out_shape=jax.ShapeDtypeStruct((N_NODES, D_FEAT), jnp.float32),
        scratch_shapes=[
            pltpu.VMEM((N_GRAPHS, D_FEAT), jnp.float32),
            pltpu.VMEM((N_GRAPHS, D_FEAT), jnp.float32),
        ],
    )(ids3, node_features, sums, sumsq, cnt, a2, b2, g2)
    return out
